# baseline (speedup 1.0000x reference)
import functools
import math

import jax
import jax.numpy as jnp
from jax import lax
from jax.experimental import pallas as pl
from jax.experimental.pallas import tpu as pltpu
from jax.experimental.pallas import tpu_sc as plsc

REHEARSAL_WEIGHT = 0.5

_NSUB = 16
_CHUNKS = 8
_CHROWS = 256

_DNUMS = lax.GatherDimensionNumbers(offset_dims=(), collapsed_slice_dims=(0,),
                                    start_index_map=(0,))


def _rot(v, lane, sh):
    perm = lax.rem(lane + sh, jnp.full((16,), 16, jnp.int32))
    return lax.gather(v, perm[:, None], _DNUMS, slice_sizes=(1,),
                      mode=lax.GatherScatterMode.PROMISE_IN_BOUNDS)


def _vsum(v, lane):
    for sh in (8, 4, 2, 1):
        v = v + _rot(v, lane, sh)
    return v


def _vmax(v, lane):
    for sh in (8, 4, 2, 1):
        v = jnp.maximum(v, _rot(v, lane, sh))
    return v


def _ln(x):
    bits = lax.bitcast_convert_type(x, jnp.int32)
    e = (((bits >> 23) & 0xFF) - 127).astype(jnp.float32)
    m = lax.bitcast_convert_type((bits & 0x7FFFFF) | 0x3F800000, jnp.float32)
    t = (m - 1.0) / (m + 1.0)
    t2 = t * t
    ln_m = 2.0 * t * (1.0 + t2 * (1.0 / 3.0 + t2 * (0.2 + t2 * (1.0 / 7.0))))
    return e * jnp.float32(0.6931471805599453) + ln_m


def _gating_loss_kernel(x_hbm, hc_hbm, smk_hbm, out_hbm,
                        buf_a, buf_b, sem_a, sem_b, stage, shared, comb,
                        hc_v, smk_v, out_v, *, n_rows, k):
    sid = lax.axis_index("s")
    bufs = (buf_a, buf_b)
    sems = (sem_a, sem_b)

    row0 = sid * (_CHUNKS * _CHROWS)

    def mk(ch):
        return pltpu.make_async_copy(
            x_hbm.at[pl.ds(row0 + ch * _CHROWS, _CHROWS), :],
            bufs[ch % 2],
            sems[ch % 2],
        )

    mk(0).start()
    mk(1).start()

    zeros16 = jnp.zeros((16,), jnp.float32)
    accs = (zeros16, zeros16, zeros16, zeros16)
    for ch in range(_CHUNKS):
        mk(ch).wait()
        buf = bufs[ch % 2]

        def body(r, carry, buf=buf):
            return tuple(
                carry[g] + buf[r, pl.ds(g * 16, 16)] for g in range(4)
            )

        accs = lax.fori_loop(0, _CHROWS, body, accs)
        if ch + 2 < _CHUNKS:
            mk(ch + 2).start()

    for g in range(4):
        stage[pl.ds(g * 16, 16)] = accs[g]
    pltpu.sync_copy(stage, shared.at[sid])
    plsc.subcore_barrier()

    @pl.when(sid == 0)
    def _epilogue():
        pltpu.sync_copy(shared, comb)
        pltpu.sync_copy(hc_hbm, hc_v)
        pltpu.sync_copy(smk_hbm, smk_v)

        lane = lax.broadcasted_iota(jnp.int32, (16,), 0)
        lane_f = lane.astype(jnp.float32)

        cs = []
        for g in range(4):
            a = comb[0, pl.ds(g * 16, 16)]
            for w in range(1, _NSUB):
                a = a + comb[w, pl.ds(g * 16, 16)]
            cs.append(a)

        idxv = smk_v[...].astype(jnp.float32)
        sel = [jnp.zeros((16,), jnp.float32) for _ in range(4)]

        def eqf(a, b):
            d = a - b
            return jnp.maximum(0.0, 1.0 - d * d)

        for j in range(k):
            onehot = eqf(lane_f, float(j))
            sj = _vsum(idxv * onehot, lane)
            for g in range(4):
                ids_f = lane_f + float(g * 16)
                sel[g] = jnp.maximum(sel[g], eqf(ids_f, sj))

        scount = zeros16
        ssum = zeros16
        for g in range(4):
            scount = scount + _vsum(sel[g], lane)
            ssum = ssum + _vsum(sel[g] * cs[g], lane)

        inv_n = 1.0 / n_rows
        log_inv_k = -math.log(float(k))
        smk_loss = scount * (1.0 / k) * log_inv_k - (1.0 / k) * ssum * inv_n

        hcg = [hc_v[pl.ds(g * 16, 16)] for g in range(4)]
        cg = [jnp.minimum(jnp.maximum(h, -10.0), 10.0) for h in hcg]
        m = _vmax(cg[0], lane)
        for g in range(1, 4):
            m = jnp.maximum(m, _vmax(cg[g], lane))
        eg = [jnp.exp(c - m) for c in cg]
        ssum_exp = zeros16
        for g in range(4):
            ssum_exp = ssum_exp + _vsum(eg[g], lane)
        lse = m + _ln(ssum_exp)

        pr_r = zeros16
        pr_cs = zeros16
        abs_hc = zeros16
        for g in range(4):
            p = eg[g] / ssum_exp
            pr_r = pr_r + _vsum(p * (cg[g] - lse), lane)
            pr_cs = pr_cs + _vsum(p * cs[g], lane)
            abs_hc = abs_hc + _vsum(jnp.abs(hcg[g]), lane)
        rehearsal_loss = pr_r - pr_cs * inv_n

        flag = jnp.minimum(abs_hc * jnp.float32(1e38), 1.0)
        loss = (smk_loss * (1.0 - REHEARSAL_WEIGHT * flag)
                + REHEARSAL_WEIGHT * flag * rehearsal_loss)
        out_v[...] = loss
        pltpu.sync_copy(out_v, out_hbm)


def kernel(log_probs, history_context, smk_indices):
    B, T, E = log_probs.shape
    n_rows = B * T
    k = smk_indices.shape[0]

    mesh = plsc.VectorSubcoreMesh(core_axis_name="c", subcore_axis_name="s",
                                  num_cores=1, num_subcores=_NSUB)

    f = pl.kernel(
        functools.partial(_gating_loss_kernel, n_rows=n_rows, k=k),
        out_type=jax.ShapeDtypeStruct((16,), jnp.float32),
        mesh=mesh,
        scratch_types=[
            pltpu.VMEM((_CHROWS, E), jnp.float32),
            pltpu.VMEM((_CHROWS, E), jnp.float32),
            pltpu.SemaphoreType.DMA,
            pltpu.SemaphoreType.DMA,
            pltpu.VMEM((E,), jnp.float32),
            pltpu.VMEM_SHARED((_NSUB, E), jnp.float32),
            pltpu.VMEM((_NSUB, E), jnp.float32),
            pltpu.VMEM((E,), jnp.float32),
            pltpu.VMEM((16,), jnp.int32),
            pltpu.VMEM((16,), jnp.float32),
        ],
    )
    smk16 = jnp.full((16,), -1, jnp.int32).at[:k].set(smk_indices)
    out = f(log_probs.reshape(n_rows, E), history_context, smk16)
    return out[0]


# speedup vs baseline: 1.0437x; 1.0437x over previous
import functools
import math

import jax
import jax.numpy as jnp
from jax import lax
from jax.experimental import pallas as pl
from jax.experimental.pallas import tpu as pltpu
from jax.experimental.pallas import tpu_sc as plsc

REHEARSAL_WEIGHT = 0.5

_NSUB = 16
_CHUNKS = 8
_CHROWS = 256

_DNUMS = lax.GatherDimensionNumbers(offset_dims=(), collapsed_slice_dims=(0,),
                                    start_index_map=(0,))


def _rot(v, lane, sh):
    perm = lax.rem(lane + sh, jnp.full((16,), 16, jnp.int32))
    return lax.gather(v, perm[:, None], _DNUMS, slice_sizes=(1,),
                      mode=lax.GatherScatterMode.PROMISE_IN_BOUNDS)


def _vsum(v, lane):
    for sh in (8, 4, 2, 1):
        v = v + _rot(v, lane, sh)
    return v


def _vmax(v, lane):
    for sh in (8, 4, 2, 1):
        v = jnp.maximum(v, _rot(v, lane, sh))
    return v


def _ln(x):
    bits = lax.bitcast_convert_type(x, jnp.int32)
    e = (((bits >> 23) & 0xFF) - 127).astype(jnp.float32)
    m = lax.bitcast_convert_type((bits & 0x7FFFFF) | 0x3F800000, jnp.float32)
    t = (m - 1.0) / (m + 1.0)
    t2 = t * t
    ln_m = 2.0 * t * (1.0 + t2 * (1.0 / 3.0 + t2 * (0.2 + t2 * (1.0 / 7.0))))
    return e * jnp.float32(0.6931471805599453) + ln_m


def _gating_loss_kernel(x_hbm, hc_hbm, smk_hbm, out_hbm,
                        buf_a, buf_b, sem_a, sem_b, stage, shared, comb,
                        hc_v, smk_v, out_v, *, n_rows, k):
    sid = lax.axis_index("s")
    bufs = (buf_a, buf_b)
    sems = (sem_a, sem_b)

    b = sid // 4
    row0 = (sid % 4) * (_CHUNKS * _CHROWS)

    def mk(ch):
        return pltpu.make_async_copy(
            x_hbm.at[b, pl.ds(row0 + ch * _CHROWS, _CHROWS), :],
            bufs[ch % 2],
            sems[ch % 2],
        )

    mk(0).start()
    mk(1).start()

    zeros16 = jnp.zeros((16,), jnp.float32)
    accs = (zeros16, zeros16, zeros16, zeros16)
    for ch in range(_CHUNKS):
        mk(ch).wait()
        buf = bufs[ch % 2]

        def body(i, carry, buf=buf):
            a = list(carry)
            for u in range(4):
                r = i * 4 + u
                for g in range(4):
                    a[g] = a[g] + buf[r, pl.ds(g * 16, 16)]
            return tuple(a)

        accs = lax.fori_loop(0, _CHROWS // 4, body, accs)
        if ch + 2 < _CHUNKS:
            mk(ch + 2).start()

    for g in range(4):
        stage[pl.ds(g * 16, 16)] = accs[g]
    pltpu.sync_copy(stage, shared.at[sid])
    plsc.subcore_barrier()

    @pl.when(sid == 0)
    def _epilogue():
        pltpu.sync_copy(shared, comb)
        pltpu.sync_copy(hc_hbm, hc_v)
        pltpu.sync_copy(smk_hbm, smk_v)

        lane = lax.broadcasted_iota(jnp.int32, (16,), 0)
        lane_f = lane.astype(jnp.float32)

        cs = []
        for g in range(4):
            a = comb[0, pl.ds(g * 16, 16)]
            for w in range(1, _NSUB):
                a = a + comb[w, pl.ds(g * 16, 16)]
            cs.append(a)

        idxv = smk_v[...].astype(jnp.float32)
        sel = [jnp.zeros((16,), jnp.float32) for _ in range(4)]

        def eqf(a, b):
            d = a - b
            return jnp.maximum(0.0, 1.0 - d * d)

        for j in range(k):
            onehot = eqf(lane_f, float(j))
            sj = _vsum(idxv * onehot, lane)
            for g in range(4):
                ids_f = lane_f + float(g * 16)
                sel[g] = jnp.maximum(sel[g], eqf(ids_f, sj))

        scount = zeros16
        ssum = zeros16
        for g in range(4):
            scount = scount + _vsum(sel[g], lane)
            ssum = ssum + _vsum(sel[g] * cs[g], lane)

        inv_n = 1.0 / n_rows
        log_inv_k = -math.log(float(k))
        smk_loss = scount * (1.0 / k) * log_inv_k - (1.0 / k) * ssum * inv_n

        hcg = [hc_v[pl.ds(g * 16, 16)] for g in range(4)]
        cg = [jnp.minimum(jnp.maximum(h, -10.0), 10.0) for h in hcg]
        m = _vmax(cg[0], lane)
        for g in range(1, 4):
            m = jnp.maximum(m, _vmax(cg[g], lane))
        eg = [jnp.exp(c - m) for c in cg]
        ssum_exp = zeros16
        for g in range(4):
            ssum_exp = ssum_exp + _vsum(eg[g], lane)
        lse = m + _ln(ssum_exp)

        pr_r = zeros16
        pr_cs = zeros16
        abs_hc = zeros16
        for g in range(4):
            p = eg[g] / ssum_exp
            pr_r = pr_r + _vsum(p * (cg[g] - lse), lane)
            pr_cs = pr_cs + _vsum(p * cs[g], lane)
            abs_hc = abs_hc + _vsum(jnp.abs(hcg[g]), lane)
        rehearsal_loss = pr_r - pr_cs * inv_n

        flag = jnp.minimum(abs_hc * jnp.float32(1e38), 1.0)
        loss = (smk_loss * (1.0 - REHEARSAL_WEIGHT * flag)
                + REHEARSAL_WEIGHT * flag * rehearsal_loss)
        out_v[...] = loss
        pltpu.sync_copy(out_v, out_hbm)


def kernel(log_probs, history_context, smk_indices):
    B, T, E = log_probs.shape
    n_rows = B * T
    k = smk_indices.shape[0]

    mesh = plsc.VectorSubcoreMesh(core_axis_name="c", subcore_axis_name="s",
                                  num_cores=1, num_subcores=_NSUB)

    f = pl.kernel(
        functools.partial(_gating_loss_kernel, n_rows=n_rows, k=k),
        out_type=jax.ShapeDtypeStruct((16,), jnp.float32),
        mesh=mesh,
        scratch_types=[
            pltpu.VMEM((_CHROWS, E), jnp.float32),
            pltpu.VMEM((_CHROWS, E), jnp.float32),
            pltpu.SemaphoreType.DMA,
            pltpu.SemaphoreType.DMA,
            pltpu.VMEM((E,), jnp.float32),
            pltpu.VMEM_SHARED((_NSUB, E), jnp.float32),
            pltpu.VMEM((_NSUB, E), jnp.float32),
            pltpu.VMEM((E,), jnp.float32),
            pltpu.VMEM((16,), jnp.int32),
            pltpu.VMEM((16,), jnp.float32),
        ],
    )
    smk16 = jnp.full((16,), -1, jnp.int32).at[:k].set(smk_indices)
    out = f(log_probs, history_context, smk16)
    return out[0]


# speedup vs baseline: 2.3136x; 2.2166x over previous
import functools

import jax
import jax.numpy as jnp
from jax.experimental import pallas as pl
from jax.experimental.pallas import tpu as pltpu

REHEARSAL_WEIGHT = 0.5


def _tree_sum_rows(chunk, rows, E):
    z = chunk.reshape(rows // 8, 8, E)
    vals = [z[j] for j in range(rows // 8)]
    while len(vals) > 1:
        nxt = [a + b for a, b in zip(vals[0::2], vals[1::2])]
        if len(vals) % 2:
            nxt[-1] = nxt[-1] + vals[-1]
        vals = nxt
    return vals[0]


def _gating_loss_kernel(x_hbm, hc_ref, smk_ref, out_ref, buf, sems, *,
                        n_chunks, chunk_rows, n_rows, k):
    E = x_hbm.shape[2]
    T = x_hbm.shape[1]
    chunks_per_batch = T // chunk_rows

    copies = []
    for c in range(n_chunks):
        b = c // chunks_per_batch
        t = c % chunks_per_batch
        cp = pltpu.make_async_copy(
            x_hbm.at[b, pl.ds(t * chunk_rows, chunk_rows), :],
            buf.at[c],
            sems.at[c],
        )
        cp.start()
        copies.append(cp)

    acc = None
    for c in range(n_chunks):
        copies[c].wait()
        p = _tree_sum_rows(buf[c], chunk_rows, E)
        acc = p if acc is None else acc + p

    colsum = jnp.sum(acc, axis=0, keepdims=True)
    hc = hc_ref[...].reshape(1, E)

    expert_ids = jax.lax.broadcasted_iota(jnp.int32, (1, E), 1)
    sel = (expert_ids == smk_ref[0]).astype(jnp.float32)
    for j in range(1, k):
        sel = jnp.maximum(sel, (expert_ids == smk_ref[j]).astype(jnp.float32))

    inv_n = 1.0 / n_rows
    log_inv_k = -jnp.log(float(k))
    scount = jnp.sum(sel)
    ssum = jnp.sum(sel * colsum)
    smk_loss = scount * (1.0 / k) * log_inv_k - (1.0 / k) * ssum * inv_n

    clamped = jnp.clip(hc, -10.0, 10.0)
    m = jnp.max(clamped)
    lse = m + jnp.log(jnp.sum(jnp.exp(clamped - m)))
    r = clamped - lse
    p_r = jnp.exp(r)
    rehearsal_loss = jnp.sum(p_r * r) - jnp.sum(p_r * colsum) * inv_n

    use_rehearsal = jnp.sum(jnp.abs(hc)) > 0.0
    loss = jnp.where(
        use_rehearsal,
        (1.0 - REHEARSAL_WEIGHT) * smk_loss + REHEARSAL_WEIGHT * rehearsal_loss,
        smk_loss,
    )
    out_ref[0] = loss


def kernel(log_probs, history_context, smk_indices):
    B, T, E = log_probs.shape
    n_rows = B * T
    k = smk_indices.shape[0]

    chunks_per_batch = 4
    n_chunks = B * chunks_per_batch
    chunk_rows = T // chunks_per_batch

    out = pl.pallas_call(
        functools.partial(_gating_loss_kernel, n_chunks=n_chunks,
                          chunk_rows=chunk_rows, n_rows=n_rows, k=k),
        in_specs=[
            pl.BlockSpec(memory_space=pl.ANY),
            pl.BlockSpec(memory_space=pltpu.VMEM),
            pl.BlockSpec(memory_space=pltpu.SMEM),
        ],
        out_specs=pl.BlockSpec(memory_space=pltpu.SMEM),
        out_shape=jax.ShapeDtypeStruct((1,), jnp.float32),
        scratch_shapes=[
            pltpu.VMEM((n_chunks, chunk_rows, E), jnp.float32),
            pltpu.SemaphoreType.DMA((n_chunks,)),
        ],
    )(log_probs, history_context, smk_indices)
    return out[0]
